# Initial kernel scaffold; baseline (speedup 1.0000x reference)
import jax
import jax.numpy as jnp
from jax.experimental import pallas as pl


def kernel(x, edge_index, edge_weight, W1, b1, Wl, bl, Wm1, bm1, Wm2, bm2, Wm3, bm3):
    raise NotImplementedError("write your pallas kernel here")


# speedup vs baseline: 11.9698x; 11.9698x over previous
import functools

import jax
import jax.numpy as jnp
from jax import lax
from jax.experimental import pallas as pl
from jax.experimental.pallas import tpu as pltpu
from jax.experimental.pallas import tpu_sc as plsc

N = 10000
E = 320000
D = 128
NC, NS = 2, 16
NW = NC * NS
EPW = E // NW
CH = 80
NCHUNK = EPW // CH
ROWS_PER_TILE = N // NS

def _sc_mesh():
    return plsc.VectorSubcoreMesh(core_axis_name="c", subcore_axis_name="s",
                                  num_cores=NC, num_subcores=NS)


def _zero_vec16():
    return jnp.zeros((16,), jnp.float32)


def _deg_body(dst_hbm, ew_hbm, out_hbm, dstv, ewv, zbuf, deg_sh, sem):
    c = lax.axis_index("c")
    s = lax.axis_index("s")
    wid = s * NC + c

    def zb(i, _):
        zbuf[pl.ds(i * 16, 16)] = _zero_vec16()
        return 0
    lax.fori_loop(0, 2000 // 16, zb, 0)

    @pl.when(s < 5)
    def _():
        pltpu.sync_copy(zbuf, deg_sh.at[pl.ds(s * 2000, 2000)])

    plsc.subcore_barrier()

    def cbody(t, _):
        base = wid * EPW + t * CH
        pltpu.sync_copy(dst_hbm.at[pl.ds(base, CH)], dstv)
        pltpu.sync_copy(ew_hbm.at[pl.ds(base, CH)], ewv)
        pltpu.sync_copy(ewv, deg_sh.at[dstv], add=True)
        return 0
    lax.fori_loop(0, NCHUNK, cbody, 0)

    plsc.subcore_barrier()

    @pl.when(s < 5)
    def _():
        pltpu.sync_copy(deg_sh.at[pl.ds(s * 2000, 2000)], zbuf)
        pltpu.sync_copy(zbuf, out_hbm.at[pl.ds(c * N + s * 2000, 2000)])


@functools.cache
def _deg_call():
    return pl.kernel(
        _deg_body,
        out_type=jax.ShapeDtypeStruct((NC * N,), jnp.float32),
        mesh=_sc_mesh(),
        scratch_types=[
            pltpu.VMEM((CH,), jnp.int32),
            pltpu.VMEM((CH,), jnp.float32),
            pltpu.VMEM((2000,), jnp.float32),
            pltpu.VMEM_SHARED((N,), jnp.float32),
            pltpu.SemaphoreType.DMA,
        ],
    )


def _prep_body(degp_ref, x_ref, dinv_ref, xs_ref):
    deg = degp_ref[:, 0:1] + degp_ref[:, 1:2] + 1.0
    dinv = lax.rsqrt(deg)
    dinv_ref[...] = dinv
    xs_ref[...] = x_ref[...] * dinv


_PREP_BLK = 1000


def _prep_call(degp_t, x):
    grid = N // _PREP_BLK
    return pl.pallas_call(
        _prep_body,
        grid=(grid,),
        in_specs=[
            pl.BlockSpec((_PREP_BLK, 2), lambda i: (i, 0)),
            pl.BlockSpec((_PREP_BLK, D), lambda i: (i, 0)),
        ],
        out_specs=[
            pl.BlockSpec((_PREP_BLK, 1), lambda i: (i, 0)),
            pl.BlockSpec((_PREP_BLK, D), lambda i: (i, 0)),
        ],
        out_shape=[
            jax.ShapeDtypeStruct((N, 1), jnp.float32),
            jax.ShapeDtypeStruct((N, D), jnp.float32),
        ],
    )(degp_t, x)


def _agg_body(src_hbm, ew_hbm, dst_hbm, xs_hbm, out_hbm,
              srcv, ewv, dstv, rows, acc_sh, sem):
    c = lax.axis_index("c")
    s = lax.axis_index("s")
    wid = s * NC + c

    def zb(i, _):
        for j in range(8):
            rows[i, pl.ds(j * 16, 16)] = _zero_vec16()
        return 0
    lax.fori_loop(0, CH, zb, 0)

    r0 = s * 624
    for k in range(7):
        pltpu.sync_copy(rows, acc_sh.at[pl.ds(r0 + k * CH, CH)])
    pltpu.sync_copy(rows.at[pl.ds(0, 64)], acc_sh.at[pl.ds(r0 + 560, 64)])

    @pl.when(s == NS - 1)
    def _():
        pltpu.sync_copy(rows.at[pl.ds(0, 16)], acc_sh.at[pl.ds(9984, 16)])

    plsc.subcore_barrier()

    def cbody(t, _):
        base = wid * EPW + t * CH
        pltpu.sync_copy(src_hbm.at[pl.ds(base, CH)], srcv)
        pltpu.sync_copy(dst_hbm.at[pl.ds(base, CH)], dstv)
        pltpu.sync_copy(ew_hbm.at[pl.ds(base, CH)], ewv.at[pl.ds(0, CH)])
        pltpu.async_copy(xs_hbm.at[srcv], rows, sem).wait()

        def ebody(e, _):
            w = ewv[pl.ds(e, 16)][0]
            for j in range(8):
                sl = pl.ds(j * 16, 16)
                rows[e, sl] = rows[e, sl] * w
            return 0
        lax.fori_loop(0, CH, ebody, 0)

        pltpu.sync_copy(rows, acc_sh.at[dstv], add=True)
        return 0
    lax.fori_loop(0, NCHUNK, cbody, 0)

    plsc.subcore_barrier()
    for k in range(7):
        pltpu.sync_copy(acc_sh.at[pl.ds(r0 + k * CH, CH)], rows)
        pltpu.sync_copy(rows, out_hbm.at[pl.ds(c * N + r0 + k * CH, CH)])
    pltpu.sync_copy(acc_sh.at[pl.ds(r0 + 560, 64)], rows.at[pl.ds(0, 64)])
    pltpu.sync_copy(rows.at[pl.ds(0, 64)],
                    out_hbm.at[pl.ds(c * N + r0 + 560, 64)])

    @pl.when(s == NS - 1)
    def _():
        pltpu.sync_copy(acc_sh.at[pl.ds(9984, 16)], rows.at[pl.ds(0, 16)])
        pltpu.sync_copy(rows.at[pl.ds(0, 16)],
                        out_hbm.at[pl.ds(c * N + 9984, 16)])


@functools.cache
def _agg_call():
    return pl.kernel(
        _agg_body,
        out_type=jax.ShapeDtypeStruct((NC * N, D), jnp.float32),
        mesh=_sc_mesh(),
        scratch_types=[
            pltpu.VMEM((CH,), jnp.int32),
            pltpu.VMEM((CH + 16,), jnp.float32),
            pltpu.VMEM((CH,), jnp.int32),
            pltpu.VMEM((CH, D), jnp.float32),
            pltpu.VMEM_SHARED((N, D), jnp.float32),
            pltpu.SemaphoreType.DMA,
        ],
    )


def _mlp_body(dinv_ref, x_ref, xs_ref, a0_ref, a1_ref,
              W1_ref, b1_ref, Wlx_ref, Wlh_ref, bl_ref,
              Wm1_ref, bm1_ref, Wm2_ref, bm2_ref, Wm3_ref, bm3_ref, out_ref):
    f32 = jnp.float32
    dinv = dinv_ref[...]
    agg = dinv * (a0_ref[...] + a1_ref[...] + xs_ref[...])
    h = jnp.maximum(
        jnp.dot(agg, W1_ref[...], preferred_element_type=f32) + b1_ref[...], 0.0)
    z = (jnp.dot(x_ref[...], Wlx_ref[...], preferred_element_type=f32)
         + jnp.dot(h, Wlh_ref[...], preferred_element_type=f32) + bl_ref[...])
    z = jnp.maximum(z, 0.0)
    z = jnp.maximum(
        jnp.dot(z, Wm1_ref[...], preferred_element_type=f32) + bm1_ref[...], 0.0)
    z = jnp.maximum(
        jnp.dot(z, Wm2_ref[...], preferred_element_type=f32) + bm2_ref[...], 0.0)
    out_ref[...] = (jnp.dot(z, Wm3_ref[...], preferred_element_type=f32)
                    + bm3_ref[...])


_MLP_BLK = 1000


def _mlp_call(dinv, x, xs, a0, a1, W1p, b1p, Wlxp, Wlhp, blp,
              Wm1p, bm1p, Wm2p, bm2p, Wm3p, bm3p):
    grid = N // _MLP_BLK
    HP = W1p.shape[1]
    H2P = Wm1p.shape[1]
    H3P = Wm2p.shape[1]
    CP = Wm3p.shape[1]

    def row(i):
        return (i, 0)

    def fixed(i):
        return (0, 0)

    return pl.pallas_call(
        _mlp_body,
        grid=(grid,),
        in_specs=[
            pl.BlockSpec((_MLP_BLK, 1), row),
            pl.BlockSpec((_MLP_BLK, D), row),
            pl.BlockSpec((_MLP_BLK, D), row),
            pl.BlockSpec((_MLP_BLK, D), row),
            pl.BlockSpec((_MLP_BLK, D), row),
            pl.BlockSpec((D, HP), fixed),
            pl.BlockSpec((1, HP), fixed),
            pl.BlockSpec((D, HP), fixed),
            pl.BlockSpec((HP, HP), fixed),
            pl.BlockSpec((1, HP), fixed),
            pl.BlockSpec((HP, H2P), fixed),
            pl.BlockSpec((1, H2P), fixed),
            pl.BlockSpec((H2P, H3P), fixed),
            pl.BlockSpec((1, H3P), fixed),
            pl.BlockSpec((H3P, CP), fixed),
            pl.BlockSpec((1, CP), fixed),
        ],
        out_specs=pl.BlockSpec((_MLP_BLK, CP), row),
        out_shape=jax.ShapeDtypeStruct((N, CP), jnp.float32),
    )(dinv, x, xs, a0, a1, W1p, b1p, Wlxp, Wlhp, blp,
      Wm1p, bm1p, Wm2p, bm2p, Wm3p, bm3p)


def _pad2(a, r, c):
    return jnp.pad(a, ((0, r - a.shape[0]), (0, c - a.shape[1])))


def kernel(x, edge_index, edge_weight, W1, b1, Wl, bl,
           Wm1, bm1, Wm2, bm2, Wm3, bm3):
    src = edge_index[0]
    dst = edge_index[1]

    degp = _deg_call()(dst, edge_weight).reshape(NC, N)
    dinv, xs = _prep_call(degp.T, x)
    accp = _agg_call()(src, edge_weight, dst, xs).reshape(NC, N, D)

    H = W1.shape[1]
    H2 = Wm1.shape[1]
    H3 = Wm2.shape[1]
    C = Wm3.shape[1]
    HP, H2P, H3P, CP = 256, 128, 128, 128

    W1p = _pad2(W1, D, HP)
    b1p = _pad2(b1[None, :], 1, HP)
    Wlxp = _pad2(Wl[:D], D, HP)
    Wlhp = _pad2(Wl[D:], HP, HP)
    blp = _pad2(bl[None, :], 1, HP)
    Wm1p = _pad2(Wm1, HP, H2P)
    bm1p = _pad2(bm1[None, :], 1, H2P)
    Wm2p = _pad2(Wm2, H2P, H3P)
    bm2p = _pad2(bm2[None, :], 1, H3P)
    Wm3p = _pad2(Wm3, H3P, CP)
    bm3p = _pad2(bm3[None, :], 1, CP)

    out = _mlp_call(dinv, x, xs, accp[0], accp[1], W1p, b1p, Wlxp, Wlhp, blp,
                    Wm1p, bm1p, Wm2p, bm2p, Wm3p, bm3p)
    return out[:, :C]
